# baseline (speedup 1.0000x reference)
import functools
import math

import jax
import jax.numpy as jnp
from jax import lax
from jax.experimental import pallas as pl
from jax.experimental.pallas import tpu as pltpu
from jax.experimental.pallas import tpu_sc as plsc

VOCAB = 1_000_000
D = 64
B_ROWS = 4096
B_COLS = 200

NC = 2
NS = 16
NW = NC * NS
IW = B_ROWS // NW
DT = D // 8
GB = IW // 16
KB = D // 16
MP = 17
SCALE = math.sqrt(D)

_mesh = plsc.VectorSubcoreMesh(core_axis_name="c", subcore_axis_name="s")


@functools.partial(
    pl.kernel,
    out_type=jax.ShapeDtypeStruct((B_COLS, DT, NW, 8, 128), jnp.float32),
    mesh=_mesh,
    compiler_params=pltpu.CompilerParams(
        use_tc_tiling_on_sc=False, needs_layout_passes=False),
    scratch_types=[
        pltpu.VMEM((IW, B_COLS), jnp.int32),
        pltpu.VMEM((B_COLS, IW), jnp.int32),
        pltpu.VMEM((IW, D), jnp.float32),
        pltpu.VMEM((IW, D), jnp.float32),
        pltpu.VMEM((DT, 8, IW), jnp.float32),
        pltpu.VMEM((DT, 8, IW), jnp.float32),
        pltpu.VMEM((GB, 16 * MP), jnp.float32),
        pltpu.SemaphoreType.DMA,
        pltpu.SemaphoreType.DMA,
        pltpu.SemaphoreType.DMA,
        pltpu.SemaphoreType.DMA,
    ],
)
def _emb_kernel(tokens_hbm, table_hbm, out_hbm,
                idx_raw, idx_t, r0, r1, w0, w1, mini, sg0, sg1, sw0, sw1):
    wid = lax.axis_index("s") * NC + lax.axis_index("c")
    i0 = wid * IW
    iota = lax.iota(jnp.int32, 16)
    iota_p = iota * MP

    pltpu.sync_copy(tokens_hbm.at[pl.ds(i0, IW), :], idx_raw)

    @plsc.parallel_loop(0, B_COLS, step=1, unroll=2)
    def _txp_idx(j):
        jv = jnp.full((16,), j, jnp.int32)
        for g in range(GB):
            idx_t[j, pl.ds(16 * g, 16)] = plsc.load_gather(
                idx_raw, [iota + 16 * g, jv])

    def g_start(j, rbuf, sem):
        pltpu.async_copy(table_hbm.at[idx_t.at[j]], rbuf, sem)

    def g_wait(rbuf, sem):
        pltpu.make_async_copy(table_hbm.at[idx_t.at[0]], rbuf, sem).wait()

    def w_start(j, wbuf, sem):
        pltpu.async_copy(wbuf, out_hbm.at[j, :, wid], sem)

    def w_wait(wbuf, sem):
        pltpu.make_async_copy(wbuf, out_hbm.at[0, :, wid], sem).wait()

    def txp_scale(rbuf, wbuf):
        @plsc.parallel_loop(0, GB, step=1, unroll=4)
        def _grp(g):
            m = mini.at[g]
            for k in range(KB):
                for t in range(16):
                    v = rbuf[16 * g + t, pl.ds(16 * k, 16)]
                    plsc.store_scatter(m, [iota_p + t], v)
                for dd in range(16):
                    d = 16 * k + dd
                    w = m[pl.ds(MP * dd, 16)]
                    wbuf[d // 8, d % 8, pl.ds(16 * g, 16)] = w * SCALE

    bufs = ((r0, w0, sg0, sw0), (r1, w1, sg1, sw1))

    g_start(0, r0, sg0)
    g_start(1, r1, sg1)

    def step(it, carry):
        for p in range(2):
            j = 2 * it + p
            rb, wb, sg, sw = bufs[p]
            g_wait(rb, sg)

            @pl.when(it >= 1)
            def _():
                w_wait(wb, sw)

            txp_scale(rb, wb)
            w_start(j, wb, sw)

            @pl.when(j + 2 < B_COLS)
            def _():
                g_start(j + 2, rb, sg)
        return carry

    lax.fori_loop(0, B_COLS // 2, step, 0)

    for p in range(2):
        rb, wb, sg, sw = bufs[p]
        w_wait(wb, sw)


def kernel(tokens, table):
    out5 = _emb_kernel(tokens, table)
    return jnp.transpose(out5, (2, 4, 0, 1, 3)).reshape(B_ROWS, B_COLS, D)


# speedup vs baseline: 1.0970x; 1.0970x over previous
import functools
import math

import jax
import jax.numpy as jnp
from jax import lax
from jax.experimental import pallas as pl
from jax.experimental.pallas import tpu as pltpu
from jax.experimental.pallas import tpu_sc as plsc

VOCAB = 1_000_000
D = 64
B_ROWS = 4096
B_COLS = 200

NC = 2
NS = 16
NW = NC * NS
IW = B_ROWS // NW
DT = D // 8
GB = IW // 16
KB = D // 16
MP = 17
SCALE = math.sqrt(D)

_mesh = plsc.VectorSubcoreMesh(core_axis_name="c", subcore_axis_name="s")


@functools.partial(
    pl.kernel,
    out_type=jax.ShapeDtypeStruct((B_COLS, DT, NW, 8, 128), jnp.float32),
    mesh=_mesh,
    compiler_params=pltpu.CompilerParams(
        use_tc_tiling_on_sc=False, needs_layout_passes=False),
    scratch_types=[
        pltpu.VMEM((IW, B_COLS), jnp.int32),
        pltpu.VMEM((B_COLS, IW), jnp.int32),
        pltpu.VMEM((IW, D), jnp.float32),
        pltpu.VMEM((IW, D), jnp.float32),
        pltpu.VMEM((DT, 8, IW), jnp.float32),
        pltpu.VMEM((DT, 8, IW), jnp.float32),
        pltpu.VMEM((GB, 16 * MP), jnp.float32),
        pltpu.SemaphoreType.DMA,
        pltpu.SemaphoreType.DMA,
        pltpu.SemaphoreType.DMA,
        pltpu.SemaphoreType.DMA,
    ],
)
def _emb_kernel(tokens_hbm, table_hbm, out_hbm,
                idx_raw, idx_t, r0, r1, w0, w1, mini, sg0, sg1, sw0, sw1):
    wid = lax.axis_index("s") * NC + lax.axis_index("c")
    i0 = wid * IW
    iota = lax.iota(jnp.int32, 16)
    iota_p = iota * MP

    pltpu.sync_copy(tokens_hbm.at[pl.ds(i0, IW), :], idx_raw)

    @plsc.parallel_loop(0, B_COLS, step=1, unroll=2)
    def _txp_idx(j):
        jv = jnp.full((16,), j, jnp.int32)
        for g in range(GB):
            idx_t[j, pl.ds(16 * g, 16)] = plsc.load_gather(
                idx_raw, [iota + 16 * g, jv])

    def g_start(j, rbuf, sem):
        pltpu.async_copy(table_hbm.at[idx_t.at[j]], rbuf, sem)

    def g_wait(rbuf, sem):
        pltpu.make_async_copy(table_hbm.at[idx_t.at[0]], rbuf, sem).wait()

    def w_start(j, wbuf, sem):
        pltpu.async_copy(wbuf, out_hbm.at[j, :, wid], sem)

    def w_wait(wbuf, sem):
        pltpu.make_async_copy(wbuf, out_hbm.at[0, :, wid], sem).wait()

    def txp_scale(rbuf, wbuf):
        @plsc.parallel_loop(0, GB, step=1, unroll=2)
        def _grp(g):
            m = mini.at[g]
            for k in range(KB):
                for t in range(16):
                    v = rbuf[16 * g + t, pl.ds(16 * k, 16)]
                    plsc.store_scatter(m, [iota_p + t], v)
                for dd in range(16):
                    d = 16 * k + dd
                    w = m[pl.ds(MP * dd, 16)]
                    wbuf[d // 8, d % 8, pl.ds(16 * g, 16)] = w * SCALE

    bufs = ((r0, w0, sg0, sw0), (r1, w1, sg1, sw1))

    g_start(0, r0, sg0)
    g_start(1, r1, sg1)

    def step(it, carry):
        for p in range(2):
            j = 2 * it + p
            rb, wb, sg, sw = bufs[p]
            g_wait(rb, sg)

            @pl.when(it >= 1)
            def _():
                w_wait(wb, sw)

            txp_scale(rb, wb)
            w_start(j, wb, sw)

            @pl.when(j + 2 < B_COLS)
            def _():
                g_start(j + 2, rb, sg)
        return carry

    lax.fori_loop(0, B_COLS // 2, step, 0)

    for p in range(2):
        rb, wb, sg, sw = bufs[p]
        w_wait(wb, sw)


def kernel(tokens, table):
    out5 = _emb_kernel(tokens, table)
    return jnp.transpose(out5, (2, 4, 0, 1, 3)).reshape(B_ROWS, B_COLS, D)
